# baseline (speedup 1.0000x reference)
import functools

import jax
import jax.numpy as jnp
from jax import lax
from jax.experimental import pallas as pl
from jax.experimental.pallas import tpu as pltpu
from jax.experimental.pallas import tpu_sc as plsc


def _make_sc_gather(batch, d2):
    info = plsc.get_sparse_core_info()
    nc, ns = info.num_cores, info.num_subcores
    nw = nc * ns
    b_per_w = batch // nw
    mesh = plsc.VectorSubcoreMesh(core_axis_name="c", subcore_axis_name="s")

    @functools.partial(
        pl.kernel,
        mesh=mesh,
        out_type=jax.ShapeDtypeStruct((batch, d2), jnp.float32),
        scratch_types=[
            pltpu.VMEM((b_per_w,), jnp.int32),
            pltpu.VMEM((b_per_w, d2), jnp.float32),
            pltpu.SemaphoreType.DMA,
        ],
    )
    def gather_k(table_hbm, idx_hbm, out_hbm, idx_v, rows_v, sem):
        wid = lax.axis_index("s") * nc + lax.axis_index("c")
        base = wid * b_per_w
        pltpu.sync_copy(idx_hbm.at[pl.ds(base, b_per_w)], idx_v)
        pltpu.async_copy(table_hbm.at[idx_v], rows_v, sem).wait()
        pltpu.sync_copy(rows_v, out_hbm.at[pl.ds(base, b_per_w)])

    return gather_k


def _film_body(gb_ref, f_ref, o_ref, *, d):
    g = gb_ref[:, :d]
    b = gb_ref[:, d:]
    o_ref[...] = f_ref[...] * g[:, None, :] + b[:, None, :]


def kernel(feat, alt_idx, gamma, beta):
    batch, seq, d = feat.shape
    idx = alt_idx.astype(jnp.int32)
    table = jnp.concatenate([gamma, beta], axis=1)
    gb = _make_sc_gather(batch, 2 * d)(table, idx)

    bb = 128
    grid = (batch // bb,)
    film = pl.pallas_call(
        functools.partial(_film_body, d=d),
        grid=grid,
        in_specs=[
            pl.BlockSpec((bb, 2 * d), lambda i: (i, 0)),
            pl.BlockSpec((bb, seq, d), lambda i: (i, 0, 0)),
        ],
        out_specs=pl.BlockSpec((bb, seq, d), lambda i: (i, 0, 0)),
        out_shape=jax.ShapeDtypeStruct((batch, seq, d), jnp.float32),
        compiler_params=pltpu.CompilerParams(
            dimension_semantics=("parallel",),
        ),
    )
    return film(gb, feat)


# speedup vs baseline: 1.5194x; 1.5194x over previous
import functools

import jax
import jax.numpy as jnp
from jax import lax
from jax.experimental import pallas as pl
from jax.experimental.pallas import tpu as pltpu
from jax.experimental.pallas import tpu_sc as plsc


def _make_sc_gather(batch, d2):
    info = plsc.get_sparse_core_info()
    nc, ns = info.num_cores, info.num_subcores
    nw = nc * ns
    b_per_w = batch // nw
    mesh = plsc.VectorSubcoreMesh(core_axis_name="c", subcore_axis_name="s")

    @functools.partial(
        pl.kernel,
        mesh=mesh,
        out_type=jax.ShapeDtypeStruct((batch, d2), jnp.float32),
        scratch_types=[
            pltpu.VMEM((b_per_w,), jnp.int32),
            pltpu.VMEM((b_per_w, d2), jnp.float32),
            pltpu.SemaphoreType.DMA,
        ],
    )
    def gather_k(table_hbm, idx_hbm, out_hbm, idx_v, rows_v, sem):
        wid = lax.axis_index("s") * nc + lax.axis_index("c")
        base = wid * b_per_w
        pltpu.sync_copy(idx_hbm.at[pl.ds(base, b_per_w)], idx_v)
        pltpu.async_copy(table_hbm.at[idx_v], rows_v, sem).wait()
        pltpu.sync_copy(rows_v, out_hbm.at[pl.ds(base, b_per_w)])

    return gather_k


def _film_body(gb_ref, f_ref, o_ref, *, lanes):
    g = gb_ref[:, :lanes]
    b = gb_ref[:, lanes:]
    o_ref[...] = f_ref[...] * g[:, None, :] + b[:, None, :]


def kernel(feat, alt_idx, gamma, beta):
    batch, seq, d = feat.shape
    idx = alt_idx.astype(jnp.int32)
    lanes = 128
    rep = lanes // d
    table = jnp.concatenate([gamma] * rep + [beta] * rep, axis=1)
    gb = _make_sc_gather(batch, 2 * lanes)(table, idx)

    m = seq * d // lanes
    feat3 = feat.reshape(batch, m, lanes)

    bb = 128
    grid = (batch // bb,)
    film = pl.pallas_call(
        functools.partial(_film_body, lanes=lanes),
        grid=grid,
        in_specs=[
            pl.BlockSpec((bb, 2 * lanes), lambda i: (i, 0)),
            pl.BlockSpec((bb, m, lanes), lambda i: (i, 0, 0)),
        ],
        out_specs=pl.BlockSpec((bb, m, lanes), lambda i: (i, 0, 0)),
        out_shape=jax.ShapeDtypeStruct((batch, m, lanes), jnp.float32),
        compiler_params=pltpu.CompilerParams(
            dimension_semantics=("parallel",),
        ),
    )
    return film(gb, feat3).reshape(batch, seq, d)


# speedup vs baseline: 5.7430x; 3.7798x over previous
import functools

import jax
import jax.numpy as jnp
from jax import lax
from jax.experimental import pallas as pl
from jax.experimental.pallas import tpu as pltpu
from jax.experimental.pallas import tpu_sc as plsc


def _make_sc_gather_t(batch, rows, n):
    info = plsc.get_sparse_core_info()
    nc, ns = info.num_cores, info.num_subcores
    nw = nc * ns
    b_per_w = batch // nw
    groups = b_per_w // 16
    mesh = plsc.VectorSubcoreMesh(core_axis_name="c", subcore_axis_name="s")

    @functools.partial(
        pl.kernel,
        mesh=mesh,
        out_type=jax.ShapeDtypeStruct((nw, rows, b_per_w), jnp.float32),
        scratch_types=[
            pltpu.VMEM((b_per_w,), jnp.int32),
            pltpu.VMEM((rows * 16,), jnp.float32),
            pltpu.VMEM((rows, b_per_w), jnp.float32),
        ],
    )
    def gather_k(table_hbm, idx_hbm, out_hbm, idx_v, tab_v, out_v):
        wid = lax.axis_index("s") * nc + lax.axis_index("c")
        base = wid * b_per_w
        pltpu.sync_copy(idx_hbm.at[pl.ds(base, b_per_w)], idx_v)
        pltpu.sync_copy(table_hbm, tab_v)
        masks = []
        for g in range(groups):
            idxg = idx_v[pl.ds(g * 16, 16)]
            masks.append([idxg == j for j in range(n - 1)])

        def body(r, carry):
            v = tab_v[pl.ds(r * 16, 16)]
            vals_n = [v[j] for j in range(n)]
            for g in range(groups):
                sel = jnp.full((16,), vals_n[n - 1], jnp.float32)
                for j in range(n - 2, -1, -1):
                    sel = jnp.where(masks[g][j], vals_n[j], sel)
                out_v[r, pl.ds(g * 16, 16)] = sel
            return carry

        lax.fori_loop(0, rows, body, 0)
        pltpu.sync_copy(out_v, out_hbm.at[wid])

    return gather_k


def _film_body(gb_ref, f_ref, o_ref, *, d):
    g = gb_ref[0, :d, :]
    b = gb_ref[0, d:, :]
    o_ref[...] = f_ref[...] * g[None] + b[None]


def kernel(feat, alt_idx, gamma, beta):
    batch, seq, d = feat.shape
    n = gamma.shape[0]
    idx = alt_idx.astype(jnp.int32)
    rows = 2 * d
    tab = jnp.concatenate([gamma.T, beta.T], axis=0)
    table_flat = jnp.pad(tab, ((0, 0), (0, 16 - n))).reshape(rows * 16)
    gbt = _make_sc_gather_t(batch, rows, n)(table_flat, idx)

    nw, _, bb = gbt.shape
    feat_t = feat.transpose(1, 2, 0)
    film = pl.pallas_call(
        functools.partial(_film_body, d=d),
        grid=(batch // bb,),
        in_specs=[
            pl.BlockSpec((1, rows, bb), lambda j: (j, 0, 0)),
            pl.BlockSpec((seq, d, bb), lambda j: (0, 0, j)),
        ],
        out_specs=pl.BlockSpec((seq, d, bb), lambda j: (0, 0, j)),
        out_shape=jax.ShapeDtypeStruct((seq, d, batch), jnp.float32),
        compiler_params=pltpu.CompilerParams(
            dimension_semantics=("parallel",),
        ),
    )
    out_t = film(gbt, feat_t)
    return out_t.transpose(2, 0, 1)
